# baseline (speedup 1.0000x reference)
import functools

import jax
import jax.numpy as jnp
from jax import lax
from jax.experimental import pallas as pl
from jax.experimental.pallas import tpu as pltpu

B, N, M = 4, 16384, 512
K1 = 64
K2 = 16

_DN = 2048


def _dist_body(node_ref, x_ref, o_ref):
    n = node_ref[0]
    xx = x_ref[0]
    a2 = jnp.sum(n * n, axis=0)
    b2 = jnp.sum(xx * xx, axis=0)
    cross = jnp.zeros((n.shape[1], xx.shape[1]), jnp.float32)
    for c in range(3):
        cross = cross + n[c][:, None] * xx[c][None, :]
    d = a2[:, None] + b2[None, :] - 2.0 * cross
    o_ref[0] = jnp.maximum(d, 0.0)


def _pair_dist(node, x, nb):
    Bb, _, Mq = node.shape
    Nn = x.shape[2]
    return pl.pallas_call(
        _dist_body,
        grid=(Bb, Nn // nb),
        in_specs=[
            pl.BlockSpec((1, 3, Mq), lambda b, j: (b, 0, 0)),
            pl.BlockSpec((1, 3, nb), lambda b, j: (b, 0, j)),
        ],
        out_specs=pl.BlockSpec((1, Mq, nb), lambda b, j: (b, 0, j)),
        out_shape=jax.ShapeDtypeStruct((Bb, Mq, Nn), jnp.float32),
    )(node, x)


_MB1 = 128


def _stack1_body(xk_ref, node_ref, W1r, b1r, W2r, b2r, W3r, b3r, W4r, b4r,
                 W5r, b5r, o_ref):
    mb = node_ref.shape[2]
    P = mb * K1
    xk = xk_ref[0]
    nd = node_ref[0]
    top = xk[0:3] - nd[:, :, None]
    X0 = jnp.concatenate([top, xk[3:6]], axis=0).reshape(6, P)

    def mm(W, bb, Xm, relu=True):
        y = lax.dot_general(W, Xm, (((1,), (0,)), ((), ())),
                            preferred_element_type=jnp.float32)
        y = y + bb[:, None]
        return jnp.maximum(y, 0.0) if relu else y

    h = mm(W1r[...], b1r[...], X0)
    h = mm(W2r[...], b2r[...], h)
    h = mm(W3r[...], b3r[...], h)
    hmax = jnp.max(h.reshape(64, mb, K1), axis=2)
    hb = jnp.broadcast_to(hmax[:, :, None], (64, mb, K1)).reshape(64, P)
    H = jnp.concatenate([h, hb], axis=0)
    h = mm(W4r[...], b4r[...], H)
    h = mm(W5r[...], b5r[...], h)
    o_ref[0] = jnp.max(h.reshape(128, mb, K1), axis=2)


def _stack1(xk, node, W1, b1, W2, b2, W3, b3, W4, b4, W5, b5):
    return pl.pallas_call(
        _stack1_body,
        grid=(B, M // _MB1),
        in_specs=[
            pl.BlockSpec((1, 6, _MB1, K1), lambda b, j: (b, 0, j, 0)),
            pl.BlockSpec((1, 3, _MB1), lambda b, j: (b, 0, j)),
            pl.BlockSpec((64, 6), lambda b, j: (0, 0)),
            pl.BlockSpec((64,), lambda b, j: (0,)),
            pl.BlockSpec((64, 64), lambda b, j: (0, 0)),
            pl.BlockSpec((64,), lambda b, j: (0,)),
            pl.BlockSpec((64, 64), lambda b, j: (0, 0)),
            pl.BlockSpec((64,), lambda b, j: (0,)),
            pl.BlockSpec((128, 128), lambda b, j: (0, 0)),
            pl.BlockSpec((128,), lambda b, j: (0,)),
            pl.BlockSpec((128, 128), lambda b, j: (0, 0)),
            pl.BlockSpec((128,), lambda b, j: (0,)),
        ],
        out_specs=pl.BlockSpec((1, 128, _MB1), lambda b, j: (b, 0, j)),
        out_shape=jax.ShapeDtypeStruct((B, 128, M), jnp.float32),
    )(xk, node, W1, b1, W2, b2, W3, b3, W4, b4, W5, b5)


_MB2 = 128


def _stack2_body(nc_ref, nf_ref, spm_ref, node_ref, Wk1r, bk1r, Wk2r, bk2r,
                 Wk3r, bk3r, Wa1r, ba1r, Wa2r, ba2r, Wm1r, bm1r, Wm2r, bm2r,
                 Wm3r, bm3r, kp_ref, sg_ref):
    mb = node_ref.shape[2]
    P = mb * K2
    nc = nc_ref[0].reshape(3, P)
    nf = nf_ref[0].reshape(128, P)
    G0 = jnp.concatenate([nc, nf], axis=0)

    def mm(W, bb, Xm, relu=True):
        y = lax.dot_general(W, Xm, (((1,), (0,)), ((), ())),
                            preferred_element_type=jnp.float32)
        y = y + bb[:, None]
        return jnp.maximum(y, 0.0) if relu else y

    g = mm(Wk1r[...], bk1r[...], G0)
    g = mm(Wk2r[...], bk2r[...], g)
    g = mm(Wk3r[...], bk3r[...], g)
    gmax = jnp.max(g.reshape(256, mb, K2), axis=2)
    gb = jnp.broadcast_to(gmax[:, :, None], (256, mb, K2)).reshape(256, P)
    G = jnp.concatenate([g, gb], axis=0)
    a = mm(Wa1r[...], ba1r[...], G)
    a = mm(Wa2r[...], ba2r[...], a)
    kf1 = jnp.max(a.reshape(512, mb, K2), axis=2)
    feat = jnp.concatenate([spm_ref[0], kf1], axis=0)
    y = mm(Wm1r[...], bm1r[...], feat)
    y = mm(Wm2r[...], bm2r[...], y)
    ks = mm(Wm3r[...], bm3r[...], y, relu=False)
    kp_ref[0] = ks[0:3] + node_ref[0]
    s = ks[3]
    sg_ref[0, 0] = jnp.maximum(s, 0.0) + jnp.log1p(jnp.exp(-jnp.abs(s))) + 0.001


def _stack2(nc, nf, spm, node, Wk1, bk1, Wk2, bk2, Wk3, bk3, Wa1, ba1,
            Wa2, ba2, Wm1, bm1, Wm2, bm2, Wm3, bm3):
    wspec = lambda o, c: pl.BlockSpec((o, c), lambda b, j: (0, 0))
    bspec = lambda o: pl.BlockSpec((o,), lambda b, j: (0,))
    kp, sg = pl.pallas_call(
        _stack2_body,
        grid=(B, M // _MB2),
        in_specs=[
            pl.BlockSpec((1, 3, _MB2, K2), lambda b, j: (b, 0, j, 0)),
            pl.BlockSpec((1, 128, _MB2, K2), lambda b, j: (b, 0, j, 0)),
            pl.BlockSpec((1, 128, _MB2), lambda b, j: (b, 0, j)),
            pl.BlockSpec((1, 3, _MB2), lambda b, j: (b, 0, j)),
            wspec(256, 131), bspec(256),
            wspec(256, 256), bspec(256),
            wspec(256, 256), bspec(256),
            wspec(512, 512), bspec(512),
            wspec(512, 512), bspec(512),
            wspec(512, 640), bspec(512),
            wspec(256, 512), bspec(256),
            wspec(4, 256), bspec(4),
        ],
        out_specs=[
            pl.BlockSpec((1, 3, _MB2), lambda b, j: (b, 0, j)),
            pl.BlockSpec((1, 1, _MB2), lambda b, j: (b, 0, j)),
        ],
        out_shape=[
            jax.ShapeDtypeStruct((B, 3, M), jnp.float32),
            jax.ShapeDtypeStruct((B, 1, M), jnp.float32),
        ],
    )(nc, nf, spm, node, Wk1, bk1, Wk2, bk2, Wk3, bk3, Wa1, ba1, Wa2, ba2,
      Wm1, bm1, Wm2, bm2, Wm3, bm3)
    return kp, sg.reshape(B, M)


def _gather_nbrs(feat, idx):
    Bb, C, _ = feat.shape
    Mm, Kk = idx.shape[1], idx.shape[2]
    idx2 = jnp.broadcast_to(idx.reshape(Bb, 1, Mm * Kk), (Bb, C, Mm * Kk))
    return jnp.take_along_axis(feat, idx2, axis=2).reshape(Bb, C, Mm, Kk)


def kernel(x, sn, node, W1, b1, W2, b2, W3, b3, W4, b4, W5, b5, Wk1, bk1,
           Wk2, bk2, Wk3, bk3, Wa1, ba1, Wa2, ba2, Wm1, bm1, Wm2, bm2,
           Wm3, bm3):
    dist = _pair_dist(node, x, _DN)
    _, nn_idx = lax.top_k(-dist, K1)
    x_aug = jnp.concatenate([x, sn], axis=1)
    xk = _gather_nbrs(x_aug, nn_idx)
    spm = _stack1(xk, node, W1, b1, W2, b2, W3, b3, W4, b4, W5, b5)

    ndist = _pair_dist(node, node, M)
    _, knn_idx = lax.top_k(-ndist, K2)
    nc = _gather_nbrs(node, knn_idx) - node[:, :, :, None]
    nf = _gather_nbrs(spm, knn_idx)
    kp, sg = _stack2(nc, nf, spm, node, Wk1, bk1, Wk2, bk2, Wk3, bk3,
                     Wa1, ba1, Wa2, ba2, Wm1, bm1, Wm2, bm2, Wm3, bm3)
    return node, kp, sg


# speedup vs baseline: 10.5544x; 10.5544x over previous
import functools

import jax
import jax.numpy as jnp
from jax import lax
from jax.experimental import pallas as pl
from jax.experimental.pallas import tpu as pltpu
from jax.experimental.pallas import tpu_sc as plsc

B, N, M = 4, 16384, 512
K1 = 64
K2 = 16

_DN = 2048


def _dist_body(node_ref, x_ref, o_ref):
    n = node_ref[0]
    xx = x_ref[0]
    a2 = jnp.sum(n * n, axis=0)
    b2 = jnp.sum(xx * xx, axis=0)
    cross = jnp.zeros((n.shape[1], xx.shape[1]), jnp.float32)
    for c in range(3):
        cross = cross + n[c][:, None] * xx[c][None, :]
    d = a2[:, None] + b2[None, :] - 2.0 * cross
    o_ref[0] = jnp.maximum(d, 0.0)


def _pair_dist(node, x, nb):
    Bb, _, Mq = node.shape
    Nn = x.shape[2]
    return pl.pallas_call(
        _dist_body,
        grid=(Bb, Nn // nb),
        in_specs=[
            pl.BlockSpec((1, 3, Mq), lambda b, j: (b, 0, 0)),
            pl.BlockSpec((1, 3, nb), lambda b, j: (b, 0, j)),
        ],
        out_specs=pl.BlockSpec((1, Mq, nb), lambda b, j: (b, 0, j)),
        out_shape=jax.ShapeDtypeStruct((Bb, Mq, Nn), jnp.float32),
    )(node, x)


_MB1 = 128


def _stack1_body(xk_ref, node_ref, W1r, b1r, W2r, b2r, W3r, b3r, W4r, b4r,
                 W5r, b5r, o_ref):
    mb = node_ref.shape[2]
    P = mb * K1
    xk = xk_ref[0]
    nd = node_ref[0]
    top = xk[0:3] - nd[:, :, None]
    X0 = jnp.concatenate([top, xk[3:6]], axis=0).reshape(6, P)

    def mm(W, bb, Xm, relu=True):
        y = lax.dot_general(W, Xm, (((1,), (0,)), ((), ())),
                            preferred_element_type=jnp.float32)
        y = y + bb[:, None]
        return jnp.maximum(y, 0.0) if relu else y

    h = mm(W1r[...], b1r[...], X0)
    h = mm(W2r[...], b2r[...], h)
    h = mm(W3r[...], b3r[...], h)
    hmax = jnp.max(h.reshape(64, mb, K1), axis=2)
    hb = jnp.broadcast_to(hmax[:, :, None], (64, mb, K1)).reshape(64, P)
    H = jnp.concatenate([h, hb], axis=0)
    h = mm(W4r[...], b4r[...], H)
    h = mm(W5r[...], b5r[...], h)
    o_ref[0] = jnp.max(h.reshape(128, mb, K1), axis=2).T


def _stack1(xk, node, W1, b1, W2, b2, W3, b3, W4, b4, W5, b5):
    return pl.pallas_call(
        _stack1_body,
        grid=(B, M // _MB1),
        in_specs=[
            pl.BlockSpec((1, 16, _MB1, K1), lambda b, j: (b, 0, j, 0)),
            pl.BlockSpec((1, 3, _MB1), lambda b, j: (b, 0, j)),
            pl.BlockSpec((64, 6), lambda b, j: (0, 0)),
            pl.BlockSpec((64,), lambda b, j: (0,)),
            pl.BlockSpec((64, 64), lambda b, j: (0, 0)),
            pl.BlockSpec((64,), lambda b, j: (0,)),
            pl.BlockSpec((64, 64), lambda b, j: (0, 0)),
            pl.BlockSpec((64,), lambda b, j: (0,)),
            pl.BlockSpec((128, 128), lambda b, j: (0, 0)),
            pl.BlockSpec((128,), lambda b, j: (0,)),
            pl.BlockSpec((128, 128), lambda b, j: (0, 0)),
            pl.BlockSpec((128,), lambda b, j: (0,)),
        ],
        out_specs=pl.BlockSpec((1, _MB1, 128), lambda b, j: (b, j, 0)),
        out_shape=jax.ShapeDtypeStruct((B, M, 128), jnp.float32),
    )(xk, node, W1, b1, W2, b2, W3, b3, W4, b4, W5, b5)


_MB2 = 128


def _stack2_body(nc_ref, nf_ref, spm_ref, node_ref, Wk1r, bk1r, Wk2r, bk2r,
                 Wk3r, bk3r, Wa1r, ba1r, Wa2r, ba2r, Wm1r, bm1r, Wm2r, bm2r,
                 Wm3r, bm3r, kp_ref, sg_ref):
    mb = node_ref.shape[2]
    P = mb * K2
    nd = node_ref[0]
    nc = nc_ref[0][0:3] - nd[:, :, None]
    nf = nf_ref[0].reshape(128, P)
    G0 = jnp.concatenate([nc.reshape(3, P), nf], axis=0)

    def mm(W, bb, Xm, relu=True):
        y = lax.dot_general(W, Xm, (((1,), (0,)), ((), ())),
                            preferred_element_type=jnp.float32)
        y = y + bb[:, None]
        return jnp.maximum(y, 0.0) if relu else y

    g = mm(Wk1r[...], bk1r[...], G0)
    g = mm(Wk2r[...], bk2r[...], g)
    g = mm(Wk3r[...], bk3r[...], g)
    gmax = jnp.max(g.reshape(256, mb, K2), axis=2)
    gb = jnp.broadcast_to(gmax[:, :, None], (256, mb, K2)).reshape(256, P)
    G = jnp.concatenate([g, gb], axis=0)
    a = mm(Wa1r[...], ba1r[...], G)
    a = mm(Wa2r[...], ba2r[...], a)
    kf1 = jnp.max(a.reshape(512, mb, K2), axis=2)
    feat = jnp.concatenate([spm_ref[0], kf1], axis=0)
    y = mm(Wm1r[...], bm1r[...], feat)
    y = mm(Wm2r[...], bm2r[...], y)
    ks = mm(Wm3r[...], bm3r[...], y, relu=False)
    kp_ref[0] = ks[0:3] + node_ref[0]
    s = ks[3]
    sg_ref[0, 0] = jnp.maximum(s, 0.0) + jnp.log1p(jnp.exp(-jnp.abs(s))) + 0.001


def _stack2(nc, nf, spm, node, Wk1, bk1, Wk2, bk2, Wk3, bk3, Wa1, ba1,
            Wa2, ba2, Wm1, bm1, Wm2, bm2, Wm3, bm3):
    wspec = lambda o, c: pl.BlockSpec((o, c), lambda b, j: (0, 0))
    bspec = lambda o: pl.BlockSpec((o,), lambda b, j: (0,))
    kp, sg = pl.pallas_call(
        _stack2_body,
        grid=(B, M // _MB2),
        in_specs=[
            pl.BlockSpec((1, 16, _MB2, K2), lambda b, j: (b, 0, j, 0)),
            pl.BlockSpec((1, 128, _MB2, K2), lambda b, j: (b, 0, j, 0)),
            pl.BlockSpec((1, 128, _MB2), lambda b, j: (b, 0, j)),
            pl.BlockSpec((1, 3, _MB2), lambda b, j: (b, 0, j)),
            wspec(256, 131), bspec(256),
            wspec(256, 256), bspec(256),
            wspec(256, 256), bspec(256),
            wspec(512, 512), bspec(512),
            wspec(512, 512), bspec(512),
            wspec(512, 640), bspec(512),
            wspec(256, 512), bspec(256),
            wspec(4, 256), bspec(4),
        ],
        out_specs=[
            pl.BlockSpec((1, 3, _MB2), lambda b, j: (b, 0, j)),
            pl.BlockSpec((1, 1, _MB2), lambda b, j: (b, 0, j)),
        ],
        out_shape=[
            jax.ShapeDtypeStruct((B, 3, M), jnp.float32),
            jax.ShapeDtypeStruct((B, 1, M), jnp.float32),
        ],
    )(nc, nf, spm, node, Wk1, bk1, Wk2, bk2, Wk3, bk3, Wa1, ba1, Wa2, ba2,
      Wm1, bm1, Wm2, bm2, Wm3, bm3)
    return kp, sg.reshape(B, M)


_NW = 32


def _sc_gather(table, idx, chunk):
    Q = idx.shape[0]
    D = table.shape[1]
    qpw = Q // _NW
    nch = qpw // chunk
    mesh = plsc.VectorSubcoreMesh(core_axis_name="c", subcore_axis_name="s")

    @functools.partial(
        pl.kernel, mesh=mesh,
        compiler_params=pltpu.CompilerParams(use_tc_tiling_on_sc=False),
        out_type=jax.ShapeDtypeStruct((Q, D), jnp.float32),
        scratch_types=[
            pltpu.VMEM((qpw,), jnp.int32),
            pltpu.VMEM((chunk, D), jnp.float32),
            pltpu.VMEM((chunk, D), jnp.float32),
            pltpu.SemaphoreType.DMA,
            pltpu.SemaphoreType.DMA,
        ],
    )
    def k(table_hbm, idx_hbm, out_hbm, idx_v, rows0, rows1, sem0, sem1):
        wid = lax.axis_index("s") * 2 + lax.axis_index("c")
        base = wid * qpw
        pltpu.sync_copy(idx_hbm.at[pl.ds(base, qpw)], idx_v)
        bufs = (rows0, rows1)
        sems = (sem0, sem1)
        pltpu.async_copy(table_hbm.at[idx_v.at[pl.ds(0, chunk)]], rows0, sem0)

        def body(i, _):
            for p in range(2):
                @pl.when(lax.rem(i, 2) == p)
                def _():
                    @pl.when(i + 1 < nch)
                    def _():
                        pltpu.async_copy(
                            table_hbm.at[idx_v.at[pl.ds((i + 1) * chunk, chunk)]],
                            bufs[1 - p], sems[1 - p])
                    pltpu.make_async_copy(
                        table_hbm.at[idx_v.at[pl.ds(i * chunk, chunk)]],
                        bufs[p], sems[p]).wait()
                    pltpu.sync_copy(bufs[p],
                                    out_hbm.at[pl.ds(base + i * chunk, chunk)])
            return 0

        lax.fori_loop(0, nch, body, 0)

    return k(table, idx)


def kernel(x, sn, node, W1, b1, W2, b2, W3, b3, W4, b4, W5, b5, Wk1, bk1,
           Wk2, bk2, Wk3, bk3, Wa1, ba1, Wa2, ba2, Wm1, bm1, Wm2, bm2,
           Wm3, bm3):
    dist = _pair_dist(node, x, _DN)
    _, nn_idx = lax.top_k(-dist, K1)

    zpad = jnp.zeros((B, N, 10), jnp.float32)
    xaug_rows = jnp.concatenate(
        [x.transpose(0, 2, 1), sn.transpose(0, 2, 1), zpad],
        axis=2).reshape(B * N, 16)
    bias_n = (jnp.arange(B, dtype=jnp.int32) * N)[:, None, None]
    nn_idx_g = (nn_idx + bias_n).reshape(B * M * K1)
    xk_rows = _sc_gather(xaug_rows, nn_idx_g, 1024)
    xk_t = xk_rows.reshape(B, M * K1, 16).transpose(0, 2, 1).reshape(
        B, 16, M, K1)
    spm_rows = _stack1(xk_t, node, W1, b1, W2, b2, W3, b3, W4, b4, W5, b5)

    ndist = _pair_dist(node, node, M)
    _, knn_idx = lax.top_k(-ndist, K2)
    bias_m = (jnp.arange(B, dtype=jnp.int32) * M)[:, None, None]
    knn_idx_g = (knn_idx + bias_m).reshape(B * M * K2)

    node_rows = jnp.concatenate(
        [node.transpose(0, 2, 1), jnp.zeros((B, M, 13), jnp.float32)],
        axis=2).reshape(B * M, 16)
    nc_rows = _sc_gather(node_rows, knn_idx_g, 1024)
    nf_rows = _sc_gather(spm_rows.reshape(B * M, 128), knn_idx_g, 256)
    nc_t = nc_rows.reshape(B, M * K2, 16).transpose(0, 2, 1).reshape(
        B, 16, M, K2)
    nf_t = nf_rows.reshape(B, M * K2, 128).transpose(0, 2, 1).reshape(
        B, 128, M, K2)
    spm_cm = spm_rows.transpose(0, 2, 1)
    kp, sg = _stack2(nc_t, nf_t, spm_cm, node, Wk1, bk1, Wk2, bk2, Wk3, bk3,
                     Wa1, ba1, Wa2, ba2, Wm1, bm1, Wm2, bm2, Wm3, bm3)
    return node, kp, sg


# speedup vs baseline: 43.9922x; 4.1681x over previous
import functools

import jax
import jax.numpy as jnp
from jax import lax
from jax.experimental import pallas as pl
from jax.experimental.pallas import tpu as pltpu
from jax.experimental.pallas import tpu_sc as plsc

B, N, M = 4, 16384, 512
K1 = 64
K2 = 16

_DN = 2048


def _dist_body(node_ref, x_ref, o_ref):
    n = node_ref[0]
    xx = x_ref[0]
    a2 = jnp.sum(n * n, axis=0)
    b2 = jnp.sum(xx * xx, axis=0)
    cross = jnp.zeros((n.shape[1], xx.shape[1]), jnp.float32)
    for c in range(3):
        cross = cross + n[c][:, None] * xx[c][None, :]
    d = a2[:, None] + b2[None, :] - 2.0 * cross
    o_ref[0] = jnp.maximum(d, 0.0)


def _pair_dist(node, x, nb):
    Bb, _, Mq = node.shape
    Nn = x.shape[2]
    return pl.pallas_call(
        _dist_body,
        grid=(Bb, Nn // nb),
        in_specs=[
            pl.BlockSpec((1, 3, Mq), lambda b, j: (b, 0, 0)),
            pl.BlockSpec((1, 3, nb), lambda b, j: (b, 0, j)),
        ],
        out_specs=pl.BlockSpec((1, Mq, nb), lambda b, j: (b, 0, j)),
        out_shape=jax.ShapeDtypeStruct((Bb, Mq, Nn), jnp.float32),
    )(node, x)


_MB1 = 128


def _stack1_body(xk_ref, node_ref, W1r, b1r, W2r, b2r, W3r, b3r, W4r, b4r,
                 W5r, b5r, o_ref):
    mb = node_ref.shape[2]
    P = mb * K1
    xk = xk_ref[0]
    nd = node_ref[0]
    top = xk[0:3] - nd[:, :, None]
    X0 = jnp.concatenate([top, xk[3:6]], axis=0).reshape(6, P)

    def mm(W, bb, Xm, relu=True):
        y = lax.dot_general(W, Xm, (((1,), (0,)), ((), ())),
                            preferred_element_type=jnp.float32)
        y = y + bb[:, None]
        return jnp.maximum(y, 0.0) if relu else y

    h = mm(W1r[...], b1r[...], X0)
    h = mm(W2r[...], b2r[...], h)
    h = mm(W3r[...], b3r[...], h)
    hmax = jnp.max(h.reshape(64, mb, K1), axis=2)
    hb = jnp.broadcast_to(hmax[:, :, None], (64, mb, K1)).reshape(64, P)
    H = jnp.concatenate([h, hb], axis=0)
    h = mm(W4r[...], b4r[...], H)
    h = mm(W5r[...], b5r[...], h)
    o_ref[0] = jnp.max(h.reshape(128, mb, K1), axis=2).T


def _stack1(xk, node, W1, b1, W2, b2, W3, b3, W4, b4, W5, b5):
    return pl.pallas_call(
        _stack1_body,
        grid=(B, M // _MB1),
        in_specs=[
            pl.BlockSpec((1, 16, _MB1, K1), lambda b, j: (b, 0, j, 0)),
            pl.BlockSpec((1, 3, _MB1), lambda b, j: (b, 0, j)),
            pl.BlockSpec((64, 6), lambda b, j: (0, 0)),
            pl.BlockSpec((64,), lambda b, j: (0,)),
            pl.BlockSpec((64, 64), lambda b, j: (0, 0)),
            pl.BlockSpec((64,), lambda b, j: (0,)),
            pl.BlockSpec((64, 64), lambda b, j: (0, 0)),
            pl.BlockSpec((64,), lambda b, j: (0,)),
            pl.BlockSpec((128, 128), lambda b, j: (0, 0)),
            pl.BlockSpec((128,), lambda b, j: (0,)),
            pl.BlockSpec((128, 128), lambda b, j: (0, 0)),
            pl.BlockSpec((128,), lambda b, j: (0,)),
        ],
        out_specs=pl.BlockSpec((1, _MB1, 128), lambda b, j: (b, j, 0)),
        out_shape=jax.ShapeDtypeStruct((B, M, 128), jnp.float32),
    )(xk, node, W1, b1, W2, b2, W3, b3, W4, b4, W5, b5)


_MB2 = 128


def _stack2_body(nc_ref, nf_ref, spm_ref, node_ref, Wk1r, bk1r, Wk2r, bk2r,
                 Wk3r, bk3r, Wa1r, ba1r, Wa2r, ba2r, Wm1r, bm1r, Wm2r, bm2r,
                 Wm3r, bm3r, kp_ref, sg_ref):
    mb = node_ref.shape[2]
    P = mb * K2
    nd = node_ref[0]
    nc = nc_ref[0][0:3] - nd[:, :, None]
    nf = nf_ref[0].reshape(128, P)
    G0 = jnp.concatenate([nc.reshape(3, P), nf], axis=0)

    def mm(W, bb, Xm, relu=True):
        y = lax.dot_general(W, Xm, (((1,), (0,)), ((), ())),
                            preferred_element_type=jnp.float32)
        y = y + bb[:, None]
        return jnp.maximum(y, 0.0) if relu else y

    g = mm(Wk1r[...], bk1r[...], G0)
    g = mm(Wk2r[...], bk2r[...], g)
    g = mm(Wk3r[...], bk3r[...], g)
    gmax = jnp.max(g.reshape(256, mb, K2), axis=2)
    gb = jnp.broadcast_to(gmax[:, :, None], (256, mb, K2)).reshape(256, P)
    G = jnp.concatenate([g, gb], axis=0)
    a = mm(Wa1r[...], ba1r[...], G)
    a = mm(Wa2r[...], ba2r[...], a)
    kf1 = jnp.max(a.reshape(512, mb, K2), axis=2)
    feat = jnp.concatenate([spm_ref[0], kf1], axis=0)
    y = mm(Wm1r[...], bm1r[...], feat)
    y = mm(Wm2r[...], bm2r[...], y)
    ks = mm(Wm3r[...], bm3r[...], y, relu=False)
    kp_ref[0] = ks[0:3] + node_ref[0]
    s = ks[3]
    sg_ref[0, 0] = jnp.maximum(s, 0.0) + jnp.log1p(jnp.exp(-jnp.abs(s))) + 0.001


def _stack2(nc, nf, spm, node, Wk1, bk1, Wk2, bk2, Wk3, bk3, Wa1, ba1,
            Wa2, ba2, Wm1, bm1, Wm2, bm2, Wm3, bm3):
    wspec = lambda o, c: pl.BlockSpec((o, c), lambda b, j: (0, 0))
    bspec = lambda o: pl.BlockSpec((o,), lambda b, j: (0,))
    kp, sg = pl.pallas_call(
        _stack2_body,
        grid=(B, M // _MB2),
        in_specs=[
            pl.BlockSpec((1, 16, _MB2, K2), lambda b, j: (b, 0, j, 0)),
            pl.BlockSpec((1, 128, _MB2, K2), lambda b, j: (b, 0, j, 0)),
            pl.BlockSpec((1, 128, _MB2), lambda b, j: (b, 0, j)),
            pl.BlockSpec((1, 3, _MB2), lambda b, j: (b, 0, j)),
            wspec(256, 131), bspec(256),
            wspec(256, 256), bspec(256),
            wspec(256, 256), bspec(256),
            wspec(512, 512), bspec(512),
            wspec(512, 512), bspec(512),
            wspec(512, 640), bspec(512),
            wspec(256, 512), bspec(256),
            wspec(4, 256), bspec(4),
        ],
        out_specs=[
            pl.BlockSpec((1, 3, _MB2), lambda b, j: (b, 0, j)),
            pl.BlockSpec((1, 1, _MB2), lambda b, j: (b, 0, j)),
        ],
        out_shape=[
            jax.ShapeDtypeStruct((B, 3, M), jnp.float32),
            jax.ShapeDtypeStruct((B, 1, M), jnp.float32),
        ],
    )(nc, nf, spm, node, Wk1, bk1, Wk2, bk2, Wk3, bk3, Wa1, ba1, Wa2, ba2,
      Wm1, bm1, Wm2, bm2, Wm3, bm3)
    return kp, sg.reshape(B, M)


_NW = 32


def _sc_gather(table, idx, chunk):
    Q = idx.shape[0]
    D = table.shape[1]
    qpw = Q // _NW
    nch = qpw // chunk
    mesh = plsc.VectorSubcoreMesh(core_axis_name="c", subcore_axis_name="s")

    @functools.partial(
        pl.kernel, mesh=mesh,
        compiler_params=pltpu.CompilerParams(use_tc_tiling_on_sc=False, needs_layout_passes=False),
        out_type=jax.ShapeDtypeStruct((Q, D), jnp.float32),
        scratch_types=[
            pltpu.VMEM((qpw,), jnp.int32),
            pltpu.VMEM((chunk, D), jnp.float32),
            pltpu.VMEM((chunk, D), jnp.float32),
            pltpu.SemaphoreType.DMA,
            pltpu.SemaphoreType.DMA,
        ],
    )
    def k(table_hbm, idx_hbm, out_hbm, idx_v, rows0, rows1, sem0, sem1):
        wid = lax.axis_index("s") * 2 + lax.axis_index("c")
        base = wid * qpw
        pltpu.sync_copy(idx_hbm.at[pl.ds(base, qpw)], idx_v)
        bufs = (rows0, rows1)
        sems = (sem0, sem1)
        pltpu.async_copy(table_hbm.at[idx_v.at[pl.ds(0, chunk)]], rows0, sem0)

        def body(i, _):
            for p in range(2):
                @pl.when(lax.rem(i, 2) == p)
                def _():
                    @pl.when(i + 1 < nch)
                    def _():
                        pltpu.async_copy(
                            table_hbm.at[idx_v.at[pl.ds((i + 1) * chunk, chunk)]],
                            bufs[1 - p], sems[1 - p])
                    pltpu.make_async_copy(
                        table_hbm.at[idx_v.at[pl.ds(i * chunk, chunk)]],
                        bufs[p], sems[p]).wait()
                    pltpu.sync_copy(bufs[p],
                                    out_hbm.at[pl.ds(base + i * chunk, chunk)])
            return 0

        lax.fori_loop(0, nch, body, 0)

    return k(table, idx)


_NHIST = 2048


def _sc_topk(dist_f32, k, bias_rows):
    dist = lax.bitcast_convert_type(dist_f32, jnp.int32)
    Q, Nn = dist.shape
    qpw = Q // _NW
    NV = Nn // 16
    mpb = Q // B
    mesh = plsc.VectorSubcoreMesh(core_axis_name="c", subcore_axis_name="s")

    @functools.partial(
        pl.kernel, mesh=mesh,
        compiler_params=pltpu.CompilerParams(use_tc_tiling_on_sc=False, needs_layout_passes=False),
        out_type=jax.ShapeDtypeStruct((Q, k), jnp.int32),
        scratch_types=[
            pltpu.VMEM((Nn,), jnp.int32),
            pltpu.VMEM((Nn,), jnp.int32),
            pltpu.VMEM((_NHIST,), jnp.int32),
            pltpu.VMEM((_NHIST,), jnp.int32),
            pltpu.VMEM((k + 16,), jnp.int32),
            pltpu.VMEM((Nn + 16,), jnp.int32),
            pltpu.VMEM((Nn + 16,), jnp.int32),
            pltpu.SemaphoreType.DMA,
            pltpu.SemaphoreType.DMA,
        ],
    )
    def t(dist_hbm, out_hbm, row0, row1, hist, cum, sel, cidx, cval,
          sem0, sem1):
        wid = lax.axis_index("s") * 2 + lax.axis_index("c")
        base = wid * qpw
        lanes = lax.iota(jnp.int32, 16)
        ones = jnp.ones((16,), jnp.int32)
        zero16 = jnp.zeros((16,), jnp.int32)
        bigv = jnp.full((16,), 2**31 - 1, jnp.int32)
        bigi = jnp.full((16,), 2**31 - 1, jnp.int32)

        def vscalar(ref, pos):
            b16 = (pos // 16) * 16
            vec = ref[pl.ds(b16, 16)]
            return jnp.sum(jnp.where(lanes == pos - b16, vec,
                                     jnp.zeros_like(vec)))

        def select_row(row, q):
            bias = ((base + q) // mpb) * bias_rows

            def zed(j, _):
                hist[pl.ds(j * 16, 16)] = zero16
                return 0
            lax.fori_loop(0, _NHIST // 16, zed, 0, unroll=4)

            def hpass(i, _):
                v = jnp.maximum(row[pl.ds(i * 16, 16)], 0)
                kb = lax.shift_right_logical(v, 20)
                plsc.addupdate_scatter(hist, [kb], ones)
                return 0
            lax.fori_loop(0, NV, hpass, 0, unroll=4)

            def scan(j, carry):
                tot, nb = carry
                c = plsc.cumsum(hist[pl.ds(j * 16, 16)]) + tot
                cum[pl.ds(j * 16, 16)] = c
                nb = nb + jnp.sum((c < k).astype(jnp.int32))
                return (jnp.max(c), nb)
            _, nb = lax.fori_loop(0, _NHIST // 16, scan,
                                  (jnp.int32(0), jnp.int32(0)), unroll=4)

            cb = lax.select(nb > 0, vscalar(cum, jnp.maximum(nb - 1, 0)),
                            jnp.int32(0))

            def p2(i, carry):
                off_s, off_c = carry
                v = jnp.maximum(row[pl.ds(i * 16, 16)], 0)
                kb = lax.shift_right_logical(v, 20)
                ixg = lanes + (i * 16 + bias)
                m_lo = kb < nb
                m_eq = kb == nb
                plsc.store_compressed(sel.at[pl.ds(off_s, 16)], ixg,
                                      mask=m_lo)
                plsc.store_compressed(cidx.at[pl.ds(off_c, 16)], ixg,
                                      mask=m_eq)
                plsc.store_compressed(cval.at[pl.ds(off_c, 16)], v,
                                      mask=m_eq)
                off_s = off_s + jnp.sum(m_lo.astype(jnp.int32))
                off_c = off_c + jnp.sum(m_eq.astype(jnp.int32))
                return (off_s, off_c)
            _, off_c = lax.fori_loop(0, NV, p2,
                                     (jnp.int32(0), jnp.int32(0)), unroll=2)

            nvc = (off_c + 15) // 16

            def ext(j, _):
                def sc1(tv, carry):
                    bv, bi = carry
                    cv = cval[pl.ds(tv * 16, 16)]
                    ci = cidx[pl.ds(tv * 16, 16)]
                    valid = (lanes + tv * 16) < off_c
                    cv = jnp.where(valid, cv, bigv)
                    better = (cv < bv) | ((cv == bv) & (ci < bi))
                    return (jnp.where(better, cv, bv),
                            jnp.where(better, ci, bi))
                bv, bi = lax.fori_loop(0, nvc, sc1, (bigv, bigi))
                mv = jnp.min(bv)
                mi = jnp.min(jnp.where(bv == mv, bi, bigi))
                plsc.store_scatter(sel, [jnp.full((16,), cb + j, jnp.int32)],
                                   jnp.full((16,), mi, jnp.int32),
                                   mask=lanes == 0)

                def rm(tv, _):
                    ci = cidx[pl.ds(tv * 16, 16)]
                    cv = cval[pl.ds(tv * 16, 16)]
                    cval[pl.ds(tv * 16, 16)] = jnp.where(ci == mi, bigv, cv)
                    return 0
                lax.fori_loop(0, nvc, rm, 0)
                return 0
            lax.fori_loop(0, k - cb, ext, 0)

            pltpu.sync_copy(sel.at[pl.ds(0, k)], out_hbm.at[base + q])

        bufs = ((row0, sem0), (row1, sem1))
        pltpu.async_copy(dist_hbm.at[base], row0, sem0)

        def qbody(q, _):
            for p in range(2):
                @pl.when(lax.rem(q, 2) == p)
                def _():
                    @pl.when(q + 1 < qpw)
                    def _():
                        pltpu.async_copy(dist_hbm.at[base + q + 1],
                                         bufs[1 - p][0], bufs[1 - p][1])
                    pltpu.make_async_copy(dist_hbm.at[base + q],
                                          bufs[p][0], bufs[p][1]).wait()
                    select_row(bufs[p][0], q)
            return 0
        lax.fori_loop(0, qpw, qbody, 0)

    return t(dist)


def kernel(x, sn, node, W1, b1, W2, b2, W3, b3, W4, b4, W5, b5, Wk1, bk1,
           Wk2, bk2, Wk3, bk3, Wa1, ba1, Wa2, ba2, Wm1, bm1, Wm2, bm2,
           Wm3, bm3):
    dist = _pair_dist(node, x, _DN)
    nn_idx_g = _sc_topk(dist.reshape(B * M, N), K1, N).reshape(B * M * K1)

    zpad = jnp.zeros((B, N, 10), jnp.float32)
    xaug_rows = jnp.concatenate(
        [x.transpose(0, 2, 1), sn.transpose(0, 2, 1), zpad],
        axis=2).reshape(B * N, 16)
    xk_rows = _sc_gather(xaug_rows, nn_idx_g, 1024)
    xk_t = xk_rows.reshape(B, M * K1, 16).transpose(0, 2, 1).reshape(
        B, 16, M, K1)
    spm_rows = _stack1(xk_t, node, W1, b1, W2, b2, W3, b3, W4, b4, W5, b5)

    ndist = _pair_dist(node, node, M)
    knn_idx_g = _sc_topk(ndist.reshape(B * M, M), K2, M).reshape(B * M * K2)

    node_rows = jnp.concatenate(
        [node.transpose(0, 2, 1), jnp.zeros((B, M, 13), jnp.float32)],
        axis=2).reshape(B * M, 16)
    nc_rows = _sc_gather(node_rows, knn_idx_g, 1024)
    nf_rows = _sc_gather(spm_rows.reshape(B * M, 128), knn_idx_g, 256)
    nc_t = nc_rows.reshape(B, M * K2, 16).transpose(0, 2, 1).reshape(
        B, 16, M, K2)
    nf_t = nf_rows.reshape(B, M * K2, 128).transpose(0, 2, 1).reshape(
        B, 128, M, K2)
    spm_cm = spm_rows.transpose(0, 2, 1)
    kp, sg = _stack2(nc_t, nf_t, spm_cm, node, Wk1, bk1, Wk2, bk2, Wk3, bk3,
                     Wa1, ba1, Wa2, ba2, Wm1, bm1, Wm2, bm2, Wm3, bm3)
    return node, kp, sg


# speedup vs baseline: 45.3060x; 1.0299x over previous
import functools

import jax
import jax.numpy as jnp
from jax import lax
from jax.experimental import pallas as pl
from jax.experimental.pallas import tpu as pltpu
from jax.experimental.pallas import tpu_sc as plsc

B, N, M = 4, 16384, 512
K1 = 64
K2 = 16

_DN = 2048


def _dist_body(node_ref, x_ref, o_ref):
    n = node_ref[0]
    xx = x_ref[0]
    a2 = jnp.sum(n * n, axis=0)
    b2 = jnp.sum(xx * xx, axis=0)
    cross = jnp.zeros((n.shape[1], xx.shape[1]), jnp.float32)
    for c in range(3):
        cross = cross + n[c][:, None] * xx[c][None, :]
    d = a2[:, None] + b2[None, :] - 2.0 * cross
    o_ref[0] = jnp.maximum(d, 0.0)


def _pair_dist(node, x, nb):
    Bb, _, Mq = node.shape
    Nn = x.shape[2]
    return pl.pallas_call(
        _dist_body,
        grid=(Bb, Nn // nb),
        in_specs=[
            pl.BlockSpec((1, 3, Mq), lambda b, j: (b, 0, 0)),
            pl.BlockSpec((1, 3, nb), lambda b, j: (b, 0, j)),
        ],
        out_specs=pl.BlockSpec((1, Mq, nb), lambda b, j: (b, 0, j)),
        out_shape=jax.ShapeDtypeStruct((Bb, Mq, Nn), jnp.float32),
    )(node, x)


_MB1 = 128


def _stack1_body(xk_ref, node_ref, W1r, b1r, W2r, b2r, W3r, b3r, W4r, b4r,
                 W5r, b5r, o_ref):
    mb = node_ref.shape[2]
    P = mb * K1
    xk = xk_ref[0]
    nd = node_ref[0]
    top = xk[0:3] - nd[:, :, None]
    X0 = jnp.concatenate([top, xk[3:6]], axis=0).reshape(6, P)

    def mm(W, bb, Xm, relu=True):
        y = lax.dot_general(W, Xm, (((1,), (0,)), ((), ())),
                            preferred_element_type=jnp.float32)
        y = y + bb[:, None]
        return jnp.maximum(y, 0.0) if relu else y

    h = mm(W1r[...], b1r[...], X0)
    h = mm(W2r[...], b2r[...], h)
    h = mm(W3r[...], b3r[...], h)
    hmax = jnp.max(h.reshape(64, mb, K1), axis=2)
    hb = jnp.broadcast_to(hmax[:, :, None], (64, mb, K1)).reshape(64, P)
    H = jnp.concatenate([h, hb], axis=0)
    h = mm(W4r[...], b4r[...], H)
    h = mm(W5r[...], b5r[...], h)
    o_ref[0] = jnp.max(h.reshape(128, mb, K1), axis=2).T


def _stack1(xk, node, W1, b1, W2, b2, W3, b3, W4, b4, W5, b5):
    return pl.pallas_call(
        _stack1_body,
        grid=(B, M // _MB1),
        in_specs=[
            pl.BlockSpec((1, 16, _MB1, K1), lambda b, j: (b, 0, j, 0)),
            pl.BlockSpec((1, 3, _MB1), lambda b, j: (b, 0, j)),
            pl.BlockSpec((64, 6), lambda b, j: (0, 0)),
            pl.BlockSpec((64,), lambda b, j: (0,)),
            pl.BlockSpec((64, 64), lambda b, j: (0, 0)),
            pl.BlockSpec((64,), lambda b, j: (0,)),
            pl.BlockSpec((64, 64), lambda b, j: (0, 0)),
            pl.BlockSpec((64,), lambda b, j: (0,)),
            pl.BlockSpec((128, 128), lambda b, j: (0, 0)),
            pl.BlockSpec((128,), lambda b, j: (0,)),
            pl.BlockSpec((128, 128), lambda b, j: (0, 0)),
            pl.BlockSpec((128,), lambda b, j: (0,)),
        ],
        out_specs=pl.BlockSpec((1, _MB1, 128), lambda b, j: (b, j, 0)),
        out_shape=jax.ShapeDtypeStruct((B, M, 128), jnp.float32),
    )(xk, node, W1, b1, W2, b2, W3, b3, W4, b4, W5, b5)


_MB2 = 128


def _stack2_body(nc_ref, nf_ref, spm_ref, node_ref, Wk1r, bk1r, Wk2r, bk2r,
                 Wk3r, bk3r, Wa1r, ba1r, Wa2r, ba2r, Wm1r, bm1r, Wm2r, bm2r,
                 Wm3r, bm3r, kp_ref, sg_ref):
    mb = node_ref.shape[2]
    P = mb * K2
    nd = node_ref[0]
    nc = nc_ref[0][0:3] - nd[:, :, None]
    nf = nf_ref[0].reshape(128, P)
    G0 = jnp.concatenate([nc.reshape(3, P), nf], axis=0)

    def mm(W, bb, Xm, relu=True):
        y = lax.dot_general(W, Xm, (((1,), (0,)), ((), ())),
                            preferred_element_type=jnp.float32)
        y = y + bb[:, None]
        return jnp.maximum(y, 0.0) if relu else y

    g = mm(Wk1r[...], bk1r[...], G0)
    g = mm(Wk2r[...], bk2r[...], g)
    g = mm(Wk3r[...], bk3r[...], g)
    gmax = jnp.max(g.reshape(256, mb, K2), axis=2)
    gb = jnp.broadcast_to(gmax[:, :, None], (256, mb, K2)).reshape(256, P)
    G = jnp.concatenate([g, gb], axis=0)
    a = mm(Wa1r[...], ba1r[...], G)
    a = mm(Wa2r[...], ba2r[...], a)
    kf1 = jnp.max(a.reshape(512, mb, K2), axis=2)
    feat = jnp.concatenate([spm_ref[0], kf1], axis=0)
    y = mm(Wm1r[...], bm1r[...], feat)
    y = mm(Wm2r[...], bm2r[...], y)
    ks = mm(Wm3r[...], bm3r[...], y, relu=False)
    kp_ref[0] = ks[0:3] + node_ref[0]
    s = ks[3]
    sg_ref[0, 0] = jnp.maximum(s, 0.0) + jnp.log1p(jnp.exp(-jnp.abs(s))) + 0.001


def _stack2(nc, nf, spm, node, Wk1, bk1, Wk2, bk2, Wk3, bk3, Wa1, ba1,
            Wa2, ba2, Wm1, bm1, Wm2, bm2, Wm3, bm3):
    wspec = lambda o, c: pl.BlockSpec((o, c), lambda b, j: (0, 0))
    bspec = lambda o: pl.BlockSpec((o,), lambda b, j: (0,))
    kp, sg = pl.pallas_call(
        _stack2_body,
        grid=(B, M // _MB2),
        in_specs=[
            pl.BlockSpec((1, 16, _MB2, K2), lambda b, j: (b, 0, j, 0)),
            pl.BlockSpec((1, 128, _MB2, K2), lambda b, j: (b, 0, j, 0)),
            pl.BlockSpec((1, 128, _MB2), lambda b, j: (b, 0, j)),
            pl.BlockSpec((1, 3, _MB2), lambda b, j: (b, 0, j)),
            wspec(256, 131), bspec(256),
            wspec(256, 256), bspec(256),
            wspec(256, 256), bspec(256),
            wspec(512, 512), bspec(512),
            wspec(512, 512), bspec(512),
            wspec(512, 640), bspec(512),
            wspec(256, 512), bspec(256),
            wspec(4, 256), bspec(4),
        ],
        out_specs=[
            pl.BlockSpec((1, 3, _MB2), lambda b, j: (b, 0, j)),
            pl.BlockSpec((1, 1, _MB2), lambda b, j: (b, 0, j)),
        ],
        out_shape=[
            jax.ShapeDtypeStruct((B, 3, M), jnp.float32),
            jax.ShapeDtypeStruct((B, 1, M), jnp.float32),
        ],
    )(nc, nf, spm, node, Wk1, bk1, Wk2, bk2, Wk3, bk3, Wa1, ba1, Wa2, ba2,
      Wm1, bm1, Wm2, bm2, Wm3, bm3)
    return kp, sg.reshape(B, M)


_NW = 32


def _sc_gather(table, idx, chunk):
    Q = idx.shape[0]
    D = table.shape[1]
    qpw = Q // _NW
    nch = qpw // chunk
    mesh = plsc.VectorSubcoreMesh(core_axis_name="c", subcore_axis_name="s")

    @functools.partial(
        pl.kernel, mesh=mesh,
        compiler_params=pltpu.CompilerParams(use_tc_tiling_on_sc=False, needs_layout_passes=False),
        out_type=jax.ShapeDtypeStruct((Q, D), jnp.float32),
        scratch_types=[
            pltpu.VMEM((qpw,), jnp.int32),
            pltpu.VMEM((chunk, D), jnp.float32),
            pltpu.VMEM((chunk, D), jnp.float32),
            pltpu.SemaphoreType.DMA,
            pltpu.SemaphoreType.DMA,
        ],
    )
    def k(table_hbm, idx_hbm, out_hbm, idx_v, rows0, rows1, sem0, sem1):
        wid = lax.axis_index("s") * 2 + lax.axis_index("c")
        base = wid * qpw
        pltpu.sync_copy(idx_hbm.at[pl.ds(base, qpw)], idx_v)
        bufs = (rows0, rows1)
        sems = (sem0, sem1)
        pltpu.async_copy(table_hbm.at[idx_v.at[pl.ds(0, chunk)]], rows0, sem0)

        def body(i, _):
            for p in range(2):
                @pl.when(lax.rem(i, 2) == p)
                def _():
                    @pl.when(i + 1 < nch)
                    def _():
                        pltpu.async_copy(
                            table_hbm.at[idx_v.at[pl.ds((i + 1) * chunk, chunk)]],
                            bufs[1 - p], sems[1 - p])
                    pltpu.make_async_copy(
                        table_hbm.at[idx_v.at[pl.ds(i * chunk, chunk)]],
                        bufs[p], sems[p]).wait()
                    pltpu.sync_copy(bufs[p],
                                    out_hbm.at[pl.ds(base + i * chunk, chunk)])
            return 0

        lax.fori_loop(0, nch, body, 0)

    return k(table, idx)


def _sc_topk(dist_f32, k, bias_rows):
    dist = lax.bitcast_convert_type(dist_f32, jnp.int32)
    Q, Nn = dist.shape
    qpw = Q // _NW
    NV = Nn // 16
    mpb = Q // B
    HIST = min(2048, Nn)
    shift = 31 - (HIST.bit_length() - 1)
    mesh = plsc.VectorSubcoreMesh(core_axis_name="c", subcore_axis_name="s")

    @functools.partial(
        pl.kernel, mesh=mesh,
        compiler_params=pltpu.CompilerParams(use_tc_tiling_on_sc=False, needs_layout_passes=False),
        out_type=jax.ShapeDtypeStruct((Q, k), jnp.int32),
        scratch_types=[
            pltpu.VMEM((Nn,), jnp.int32),
            pltpu.VMEM((Nn,), jnp.int32),
            pltpu.VMEM((HIST,), jnp.int32),
            pltpu.VMEM((HIST,), jnp.int32),
            pltpu.VMEM((k + 16,), jnp.int32),
            pltpu.VMEM((Nn + 16,), jnp.int32),
            pltpu.VMEM((Nn + 16,), jnp.int32),
            pltpu.SemaphoreType.DMA,
            pltpu.SemaphoreType.DMA,
        ],
    )
    def t(dist_hbm, out_hbm, row0, row1, hist, cum, sel, cidx, cval,
          sem0, sem1):
        wid = lax.axis_index("s") * 2 + lax.axis_index("c")
        base = wid * qpw
        lanes = lax.iota(jnp.int32, 16)
        ones = jnp.ones((16,), jnp.int32)
        zero16 = jnp.zeros((16,), jnp.int32)
        bigv = jnp.full((16,), 2**31 - 1, jnp.int32)
        bigi = jnp.full((16,), 2**31 - 1, jnp.int32)

        def vscalar(ref, pos):
            b16 = (pos // 16) * 16
            vec = ref[pl.ds(b16, 16)]
            return jnp.sum(jnp.where(lanes == pos - b16, vec,
                                     jnp.zeros_like(vec)))

        def select_row(row, q):
            bias = ((base + q) // mpb) * bias_rows

            def zed(j, _):
                hist[pl.ds(j * 16, 16)] = zero16
                return 0
            lax.fori_loop(0, HIST // 16, zed, 0, unroll=4)

            def hpass(i, _):
                v = jnp.maximum(row[pl.ds(i * 16, 16)], 0)
                kb = lax.shift_right_logical(v, shift)
                plsc.addupdate_scatter(hist, [kb], ones)
                return 0
            lax.fori_loop(0, NV, hpass, 0, unroll=4)

            def scan(j, carry):
                tot, nbv = carry
                c = plsc.cumsum(hist[pl.ds(j * 16, 16)]) + tot
                cum[pl.ds(j * 16, 16)] = c
                nbv = nbv + plsc.all_reduce_population_count(c < k)
                return (jnp.max(c), nbv)
            _, nbv = lax.fori_loop(0, HIST // 16, scan,
                                   (jnp.int32(0), zero16), unroll=4)
            nb = jnp.max(nbv)

            cb = lax.select(nb > 0, vscalar(cum, jnp.maximum(nb - 1, 0)),
                            jnp.int32(0))

            def p2(i, carry):
                offs_v, offc_v = carry
                v = jnp.maximum(row[pl.ds(i * 16, 16)], 0)
                kb = lax.shift_right_logical(v, shift)
                ixg = lanes + (i * 16 + bias)
                m_lo = kb < nb
                m_eq = kb == nb
                il = m_lo.astype(jnp.int32)
                ie = m_eq.astype(jnp.int32)
                pos_s = offs_v + plsc.cumsum(il) - il
                pos_c = offc_v + plsc.cumsum(ie) - ie
                plsc.store_scatter(sel, [pos_s], ixg, mask=m_lo)
                plsc.store_scatter(cidx, [pos_c], ixg, mask=m_eq)
                plsc.store_scatter(cval, [pos_c], v, mask=m_eq)
                offs_v = offs_v + plsc.all_reduce_population_count(m_lo)
                offc_v = offc_v + plsc.all_reduce_population_count(m_eq)
                return (offs_v, offc_v)
            _, offc_v = lax.fori_loop(0, NV, p2, (zero16, zero16), unroll=4)
            off_c = jnp.max(offc_v)

            nvc = (off_c + 15) // 16

            def ext(j, _):
                def sc1(tv, carry):
                    bv, bi = carry
                    cv = cval[pl.ds(tv * 16, 16)]
                    ci = cidx[pl.ds(tv * 16, 16)]
                    valid = (lanes + tv * 16) < off_c
                    cv = jnp.where(valid, cv, bigv)
                    better = (cv < bv) | ((cv == bv) & (ci < bi))
                    return (jnp.where(better, cv, bv),
                            jnp.where(better, ci, bi))
                bv, bi = lax.fori_loop(0, nvc, sc1, (bigv, bigi))
                mv = jnp.min(bv)
                mi = jnp.min(jnp.where(bv == mv, bi, bigi))
                plsc.store_scatter(sel, [jnp.full((16,), cb + j, jnp.int32)],
                                   jnp.full((16,), mi, jnp.int32),
                                   mask=lanes == 0)

                def rm(tv, _):
                    ci = cidx[pl.ds(tv * 16, 16)]
                    cv = cval[pl.ds(tv * 16, 16)]
                    cval[pl.ds(tv * 16, 16)] = jnp.where(ci == mi, bigv, cv)
                    return 0
                lax.fori_loop(0, nvc, rm, 0)
                return 0
            lax.fori_loop(0, k - cb, ext, 0)

            pltpu.sync_copy(sel.at[pl.ds(0, k)], out_hbm.at[base + q])

        bufs = ((row0, sem0), (row1, sem1))
        pltpu.async_copy(dist_hbm.at[base], row0, sem0)

        def qbody(q, _):
            for p in range(2):
                @pl.when(lax.rem(q, 2) == p)
                def _():
                    @pl.when(q + 1 < qpw)
                    def _():
                        pltpu.async_copy(dist_hbm.at[base + q + 1],
                                         bufs[1 - p][0], bufs[1 - p][1])
                    pltpu.make_async_copy(dist_hbm.at[base + q],
                                          bufs[p][0], bufs[p][1]).wait()
                    select_row(bufs[p][0], q)
            return 0
        lax.fori_loop(0, qpw, qbody, 0)

    return t(dist)


def kernel(x, sn, node, W1, b1, W2, b2, W3, b3, W4, b4, W5, b5, Wk1, bk1,
           Wk2, bk2, Wk3, bk3, Wa1, ba1, Wa2, ba2, Wm1, bm1, Wm2, bm2,
           Wm3, bm3):
    dist = _pair_dist(node, x, _DN)
    nn_idx_g = _sc_topk(dist.reshape(B * M, N), K1, N).reshape(B * M * K1)

    zpad = jnp.zeros((B, N, 10), jnp.float32)
    xaug_rows = jnp.concatenate(
        [x.transpose(0, 2, 1), sn.transpose(0, 2, 1), zpad],
        axis=2).reshape(B * N, 16)
    xk_rows = _sc_gather(xaug_rows, nn_idx_g, 1024)
    xk_t = xk_rows.reshape(B, M * K1, 16).transpose(0, 2, 1).reshape(
        B, 16, M, K1)
    spm_rows = _stack1(xk_t, node, W1, b1, W2, b2, W3, b3, W4, b4, W5, b5)

    ndist = _pair_dist(node, node, M)
    knn_idx_g = _sc_topk(ndist.reshape(B * M, M), K2, M).reshape(B * M * K2)

    node_rows = jnp.concatenate(
        [node.transpose(0, 2, 1), jnp.zeros((B, M, 13), jnp.float32)],
        axis=2).reshape(B * M, 16)
    nc_rows = _sc_gather(node_rows, knn_idx_g, 1024)
    nf_rows = _sc_gather(spm_rows.reshape(B * M, 128), knn_idx_g, 256)
    nc_t = nc_rows.reshape(B, M * K2, 16).transpose(0, 2, 1).reshape(
        B, 16, M, K2)
    nf_t = nf_rows.reshape(B, M * K2, 128).transpose(0, 2, 1).reshape(
        B, 128, M, K2)
    spm_cm = spm_rows.transpose(0, 2, 1)
    kp, sg = _stack2(nc_t, nf_t, spm_cm, node, Wk1, bk1, Wk2, bk2, Wk3, bk3,
                     Wa1, ba1, Wa2, ba2, Wm1, bm1, Wm2, bm2, Wm3, bm3)
    return node, kp, sg
